# Initial kernel scaffold; baseline (speedup 1.0000x reference)
import jax
import jax.numpy as jnp
from jax.experimental import pallas as pl


def kernel(x, edge_index, edge_attr, Wl1, bl1, Wr1, Wl2, bl2, Wr2, Wl3, bl3, Wr3, W1, b1, W2, b2, W3, b3):
    raise NotImplementedError("write your pallas kernel here")


# speedup vs baseline: 7.6558x; 7.6558x over previous
import functools

import jax
import jax.numpy as jnp
from jax import lax
from jax.experimental import pallas as pl
from jax.experimental.pallas import tpu as pltpu
from jax.experimental.pallas import tpu_sc as plsc

N = 10000
E = 320000
H = 64

NC = 2
NS = 16
NW = NC * NS
EDGES_PER_W = E // NW
CHUNK = 1000
NP = 10240
N_PER_TILE = NP // NS

_sc_mesh = plsc.VectorSubcoreMesh(core_axis_name="c", subcore_axis_name="s")


def _wid():
    return lax.axis_index("s") * NC + lax.axis_index("c")


@functools.partial(
    pl.kernel,
    out_type=jax.ShapeDtypeStruct((NC, NP, 16), jnp.float32),
    mesh=_sc_mesh,
    compiler_params=pltpu.CompilerParams(use_tc_tiling_on_sc=False),
    scratch_types=[
        pltpu.VMEM((CHUNK,), jnp.int32),
        pltpu.VMEM((CHUNK, 16), jnp.float32),
        pltpu.VMEM_SHARED((NP, 16), jnp.float32),
    ],
)
def _sc_degree(dst_hbm, zeros_hbm, ones_hbm, out_hbm, dst_v, ones_v, acc_sh):
    cid = lax.axis_index("c")
    sid = lax.axis_index("s")
    base = _wid() * EDGES_PER_W
    pltpu.sync_copy(zeros_hbm.at[pl.ds(sid * N_PER_TILE, N_PER_TILE)],
                    acc_sh.at[pl.ds(sid * N_PER_TILE, N_PER_TILE)])
    pltpu.sync_copy(ones_hbm, ones_v)
    plsc.subcore_barrier()

    def body(k, _):
        off = base + k * CHUNK
        pltpu.sync_copy(dst_hbm.at[pl.ds(off, CHUNK)], dst_v)
        pltpu.sync_copy(ones_v, acc_sh.at[dst_v], add=True)
        return 0

    lax.fori_loop(0, EDGES_PER_W // CHUNK, body, 0)
    plsc.subcore_barrier()
    pltpu.sync_copy(acc_sh.at[pl.ds(sid * N_PER_TILE, N_PER_TILE)],
                    out_hbm.at[cid, pl.ds(sid * N_PER_TILE, N_PER_TILE)])


@functools.partial(
    pl.kernel,
    out_type=jax.ShapeDtypeStruct((NC, NP, H), jnp.float32),
    mesh=_sc_mesh,
    compiler_params=pltpu.CompilerParams(use_tc_tiling_on_sc=False),
    scratch_types=[
        pltpu.VMEM((CHUNK,), jnp.int32),
        pltpu.VMEM((CHUNK,), jnp.int32),
        pltpu.VMEM((CHUNK, H), jnp.float32),
        pltpu.VMEM_SHARED((NP, H), jnp.float32),
        pltpu.SemaphoreType.DMA,
    ],
)
def _sc_segsum(pre_hbm, src_hbm, dst_hbm, zeros_hbm, out_hbm,
               src_v, dst_v, rows_v, acc_sh, sem):
    cid = lax.axis_index("c")
    sid = lax.axis_index("s")
    base = _wid() * EDGES_PER_W
    pltpu.sync_copy(zeros_hbm.at[pl.ds(sid * N_PER_TILE, N_PER_TILE)],
                    acc_sh.at[pl.ds(sid * N_PER_TILE, N_PER_TILE)])
    plsc.subcore_barrier()

    def body(k, _):
        off = base + k * CHUNK
        pltpu.sync_copy(src_hbm.at[pl.ds(off, CHUNK)], src_v)
        pltpu.sync_copy(dst_hbm.at[pl.ds(off, CHUNK)], dst_v)
        pltpu.async_copy(pre_hbm.at[src_v], rows_v, sem).wait()
        pltpu.sync_copy(rows_v, acc_sh.at[dst_v], add=True)
        return 0

    lax.fori_loop(0, EDGES_PER_W // CHUNK, body, 0)
    plsc.subcore_barrier()
    pltpu.sync_copy(acc_sh.at[pl.ds(sid * N_PER_TILE, N_PER_TILE)],
                    out_hbm.at[cid, pl.ds(sid * N_PER_TILE, N_PER_TILE)])


@functools.partial(
    pl.kernel,
    out_type=jax.ShapeDtypeStruct((E, H), jnp.float32),
    mesh=_sc_mesh,
    compiler_params=pltpu.CompilerParams(use_tc_tiling_on_sc=False),
    scratch_types=[
        pltpu.VMEM((CHUNK,), jnp.int32),
        pltpu.VMEM((CHUNK,), jnp.int32),
        pltpu.VMEM((CHUNK, H), jnp.float32),
        pltpu.SemaphoreType.DMA,
        pltpu.SemaphoreType.DMA,
    ],
)
def _sc_edge_gather(hs_hbm, hd_hbm, src_hbm, dst_hbm, out_hbm,
                    src_v, dst_v, rows_v, sem_a, sem_b):
    base = _wid() * EDGES_PER_W

    def body(k, _):
        off = base + k * CHUNK
        pltpu.sync_copy(src_hbm.at[pl.ds(off, CHUNK)], src_v)
        pltpu.sync_copy(dst_hbm.at[pl.ds(off, CHUNK)], dst_v)
        pltpu.async_copy(hs_hbm.at[src_v], rows_v, sem_a).wait()
        pltpu.async_copy(hd_hbm.at[dst_v], rows_v, sem_b, add=True).wait()
        pltpu.sync_copy(rows_v, out_hbm.at[pl.ds(off, CHUNK)])
        return 0

    lax.fori_loop(0, EDGES_PER_W // CHUNK, body, 0)


BN = 1000


def _tc_proj_body(x_ref, wl_ref, wr_ref, pre_ref, r_ref):
    x = x_ref[...]
    pre_ref[...] = jnp.dot(x, wl_ref[...], preferred_element_type=jnp.float32)
    r_ref[...] = jnp.dot(x, wr_ref[...], preferred_element_type=jnp.float32)


def _tc_proj(x, wl_t, wr_t):
    d = x.shape[1]
    return pl.pallas_call(
        _tc_proj_body,
        grid=(N // BN,),
        in_specs=[
            pl.BlockSpec((BN, d), lambda i: (i, 0)),
            pl.BlockSpec((d, H), lambda i: (0, 0)),
            pl.BlockSpec((d, H), lambda i: (0, 0)),
        ],
        out_specs=[
            pl.BlockSpec((BN, H), lambda i: (i, 0)),
            pl.BlockSpec((BN, H), lambda i: (i, 0)),
        ],
        out_shape=[
            jax.ShapeDtypeStruct((N, H), jnp.float32),
            jax.ShapeDtypeStruct((N, H), jnp.float32),
        ],
    )(x, wl_t, wr_t)


def _tc_combine_body(acc_ref, deg_ref, r_ref, bl_ref, wa_ref, wb_ref,
                     outa_ref, outb_ref):
    deg = deg_ref[0, :, 0] + deg_ref[1, :, 0]
    inv = 1.0 / jnp.maximum(deg, 1.0)
    agg = acc_ref[0] + acc_ref[1]
    h = jnp.maximum(agg * inv[:, None] + bl_ref[0] + r_ref[...], 0.0)
    outa_ref[...] = jnp.dot(h, wa_ref[...], preferred_element_type=jnp.float32)
    outb_ref[...] = jnp.dot(h, wb_ref[...], preferred_element_type=jnp.float32)


def _tc_combine(acc, deg_acc, r, bl, wa_t, wb_t):
    return pl.pallas_call(
        _tc_combine_body,
        grid=(N // BN,),
        in_specs=[
            pl.BlockSpec((NC, BN, H), lambda i: (0, i, 0)),
            pl.BlockSpec((NC, BN, 16), lambda i: (0, i, 0)),
            pl.BlockSpec((BN, H), lambda i: (i, 0)),
            pl.BlockSpec((1, H), lambda i: (0, 0)),
            pl.BlockSpec((H, H), lambda i: (0, 0)),
            pl.BlockSpec((H, H), lambda i: (0, 0)),
        ],
        out_specs=[
            pl.BlockSpec((BN, H), lambda i: (i, 0)),
            pl.BlockSpec((BN, H), lambda i: (i, 0)),
        ],
        out_shape=[
            jax.ShapeDtypeStruct((N, H), jnp.float32),
            jax.ShapeDtypeStruct((N, H), jnp.float32),
        ],
    )(acc, deg_acc, r, bl, wa_t, wb_t)


OUT_COLS = 2000
OUT_ROWS_PER_BLK = 8
BE = OUT_COLS * OUT_ROWS_PER_BLK


def _tc_edge_mlp_body(g_ref, ea_ref, w1e_ref, b1_ref, w2_ref, b2_ref,
                      w3_ref, b3_ref, out_ref):
    z = g_ref[...] + jnp.dot(ea_ref[...], w1e_ref[...],
                             preferred_element_type=jnp.float32) + b1_ref[0]
    z = jnp.maximum(z, 0.0)
    z = jnp.maximum(jnp.dot(z, w2_ref[...],
                            preferred_element_type=jnp.float32) + b2_ref[0], 0.0)
    lg = jnp.dot(z, w3_ref[...], preferred_element_type=jnp.float32) + b3_ref[0]
    out_ref[...] = lg.reshape(OUT_ROWS_PER_BLK, OUT_COLS)


def _tc_edge_mlp(g, edge_attr, w1e_t, b1, w2_t, b2, w3_t, b3):
    out = pl.pallas_call(
        _tc_edge_mlp_body,
        grid=(E // BE,),
        in_specs=[
            pl.BlockSpec((BE, H), lambda i: (i, 0)),
            pl.BlockSpec((BE, 16), lambda i: (i, 0)),
            pl.BlockSpec((16, H), lambda i: (0, 0)),
            pl.BlockSpec((1, H), lambda i: (0, 0)),
            pl.BlockSpec((H, 32), lambda i: (0, 0)),
            pl.BlockSpec((1, 32), lambda i: (0, 0)),
            pl.BlockSpec((32, 1), lambda i: (0, 0)),
            pl.BlockSpec((1, 1), lambda i: (0, 0)),
        ],
        out_specs=pl.BlockSpec((OUT_ROWS_PER_BLK, OUT_COLS), lambda i: (i, 0)),
        out_shape=jax.ShapeDtypeStruct((E // OUT_COLS, OUT_COLS), jnp.float32),
    )(g, edge_attr, w1e_t, b1, w2_t, b2, w3_t, b3)
    return out.reshape(E)


def kernel(x, edge_index, edge_attr, Wl1, bl1, Wr1, Wl2, bl2, Wr2,
           Wl3, bl3, Wr3, W1, b1, W2, b2, W3, b3):
    src = edge_index[0]
    dst = edge_index[1]

    zeros_n64 = jnp.zeros((NP, H), jnp.float32)
    zeros_n16 = jnp.zeros((NP, 16), jnp.float32)
    ones_c16 = jnp.ones((CHUNK, 16), jnp.float32)

    deg_acc = _sc_degree(dst, zeros_n16, ones_c16)

    pre1, r1 = _tc_proj(x, Wl1.T, Wr1.T)
    acc1 = _sc_segsum(pre1, src, dst, zeros_n64)
    pre2, r2 = _tc_combine(acc1, deg_acc, r1, bl1.reshape(1, H),
                           Wl2.T, Wr2.T)
    acc2 = _sc_segsum(pre2, src, dst, zeros_n64)
    pre3, r3 = _tc_combine(acc2, deg_acc, r2, bl2.reshape(1, H),
                           Wl3.T, Wr3.T)
    acc3 = _sc_segsum(pre3, src, dst, zeros_n64)
    w1s_t = W1[:, :H].T
    w1d_t = W1[:, H:2 * H].T
    hs, hd = _tc_combine(acc3, deg_acc, r3, bl3.reshape(1, H), w1s_t, w1d_t)

    g = _sc_edge_gather(hs, hd, src, dst)

    w1e_t = W1[:, 2 * H:].T
    return _tc_edge_mlp(g, edge_attr, w1e_t, b1.reshape(1, H),
                        W2.T, b2.reshape(1, 32), W3.T, b3.reshape(1, 1))


# speedup vs baseline: 7.9147x; 1.0338x over previous
import functools

import jax
import jax.numpy as jnp
from jax import lax
from jax.experimental import pallas as pl
from jax.experimental.pallas import tpu as pltpu
from jax.experimental.pallas import tpu_sc as plsc

N = 10000
E = 320000
H = 64

NC = 2
NS = 16
NW = NC * NS
EDGES_PER_W = E // NW
CHUNK = 1000
NP = 10240
N_PER_TILE = NP // NS

_sc_mesh = plsc.VectorSubcoreMesh(core_axis_name="c", subcore_axis_name="s")


def _wid():
    return lax.axis_index("s") * NC + lax.axis_index("c")


N_TBL_SLAB = N // NS


@functools.partial(
    pl.kernel,
    out_type=jax.ShapeDtypeStruct((NC, NP, 16), jnp.float32),
    mesh=_sc_mesh,
    compiler_params=pltpu.CompilerParams(use_tc_tiling_on_sc=False),
    scratch_types=[
        pltpu.VMEM((CHUNK,), jnp.int32),
        pltpu.VMEM((CHUNK, 16), jnp.float32),
        pltpu.VMEM_SHARED((NP, 16), jnp.float32),
    ],
)
def _sc_degree(dst_hbm, zeros_hbm, ones_hbm, out_hbm, dst_v, ones_v, acc_sh):
    cid = lax.axis_index("c")
    sid = lax.axis_index("s")
    base = _wid() * EDGES_PER_W
    pltpu.sync_copy(zeros_hbm.at[pl.ds(sid * N_PER_TILE, N_PER_TILE)],
                    acc_sh.at[pl.ds(sid * N_PER_TILE, N_PER_TILE)])
    pltpu.sync_copy(ones_hbm, ones_v)
    plsc.subcore_barrier()

    def body(k, _):
        off = base + k * CHUNK
        pltpu.sync_copy(dst_hbm.at[pl.ds(off, CHUNK)], dst_v)
        pltpu.sync_copy(ones_v, acc_sh.at[dst_v], add=True)
        return 0

    lax.fori_loop(0, EDGES_PER_W // CHUNK, body, 0)
    plsc.subcore_barrier()
    pltpu.sync_copy(acc_sh.at[pl.ds(sid * N_PER_TILE, N_PER_TILE)],
                    out_hbm.at[cid, pl.ds(sid * N_PER_TILE, N_PER_TILE)])


@functools.partial(
    pl.kernel,
    out_type=jax.ShapeDtypeStruct((NC, NP, H), jnp.float32),
    mesh=_sc_mesh,
    compiler_params=pltpu.CompilerParams(use_tc_tiling_on_sc=False),
    scratch_types=[
        pltpu.VMEM((CHUNK,), jnp.int32),
        pltpu.VMEM((CHUNK,), jnp.int32),
        pltpu.VMEM((CHUNK, H), jnp.float32),
        pltpu.VMEM_SHARED((NP, H), jnp.float32),
        pltpu.SemaphoreType.DMA,
    ],
)
def _sc_segsum(pre_hbm, src_hbm, dst_hbm, zeros_hbm, out_hbm,
               src_v, dst_v, rows_v, acc_sh, sem):
    cid = lax.axis_index("c")
    sid = lax.axis_index("s")
    base = _wid() * EDGES_PER_W
    pltpu.sync_copy(zeros_hbm.at[pl.ds(sid * N_PER_TILE, N_PER_TILE)],
                    acc_sh.at[pl.ds(sid * N_PER_TILE, N_PER_TILE)])
    plsc.subcore_barrier()

    def body(k, _):
        off = base + k * CHUNK
        pltpu.sync_copy(src_hbm.at[pl.ds(off, CHUNK)], src_v)
        pltpu.sync_copy(dst_hbm.at[pl.ds(off, CHUNK)], dst_v)
        pltpu.async_copy(pre_hbm.at[src_v], rows_v, sem).wait()
        pltpu.sync_copy(rows_v, acc_sh.at[dst_v], add=True)
        return 0

    lax.fori_loop(0, EDGES_PER_W // CHUNK, body, 0)
    plsc.subcore_barrier()
    pltpu.sync_copy(acc_sh.at[pl.ds(sid * N_PER_TILE, N_PER_TILE)],
                    out_hbm.at[cid, pl.ds(sid * N_PER_TILE, N_PER_TILE)])


@functools.partial(
    pl.kernel,
    out_type=jax.ShapeDtypeStruct((E, H), jnp.float32),
    mesh=_sc_mesh,
    compiler_params=pltpu.CompilerParams(use_tc_tiling_on_sc=False),
    scratch_types=[
        pltpu.VMEM((CHUNK,), jnp.int32),
        pltpu.VMEM((CHUNK,), jnp.int32),
        pltpu.VMEM((CHUNK, H), jnp.float32),
        pltpu.VMEM_SHARED((N, H), jnp.float32),
        pltpu.SemaphoreType.DMA,
        pltpu.SemaphoreType.DMA,
    ],
)
def _sc_edge_gather(hs_hbm, hd_hbm, src_hbm, dst_hbm, out_hbm,
                    src_v, dst_v, rows_v, hs_sh, sem_a, sem_b):
    sid = lax.axis_index("s")
    base = _wid() * EDGES_PER_W
    pltpu.sync_copy(hs_hbm.at[pl.ds(sid * N_TBL_SLAB, N_TBL_SLAB)],
                    hs_sh.at[pl.ds(sid * N_TBL_SLAB, N_TBL_SLAB)])
    plsc.subcore_barrier()

    def body(k, _):
        off = base + k * CHUNK
        pltpu.sync_copy(src_hbm.at[pl.ds(off, CHUNK)], src_v)
        pltpu.sync_copy(dst_hbm.at[pl.ds(off, CHUNK)], dst_v)
        pltpu.async_copy(hs_sh.at[src_v], rows_v, sem_a).wait()
        pltpu.async_copy(hd_hbm.at[dst_v], rows_v, sem_b, add=True).wait()
        pltpu.sync_copy(rows_v, out_hbm.at[pl.ds(off, CHUNK)])
        return 0

    lax.fori_loop(0, EDGES_PER_W // CHUNK, body, 0)


BN = 1000


def _tc_proj_body(x_ref, wl_ref, wr_ref, pre_ref, r_ref):
    x = x_ref[...]
    pre_ref[...] = jnp.dot(x, wl_ref[...], preferred_element_type=jnp.float32)
    r_ref[...] = jnp.dot(x, wr_ref[...], preferred_element_type=jnp.float32)


def _tc_proj(x, wl_t, wr_t):
    d = x.shape[1]
    return pl.pallas_call(
        _tc_proj_body,
        grid=(N // BN,),
        in_specs=[
            pl.BlockSpec((BN, d), lambda i: (i, 0)),
            pl.BlockSpec((d, H), lambda i: (0, 0)),
            pl.BlockSpec((d, H), lambda i: (0, 0)),
        ],
        out_specs=[
            pl.BlockSpec((BN, H), lambda i: (i, 0)),
            pl.BlockSpec((BN, H), lambda i: (i, 0)),
        ],
        out_shape=[
            jax.ShapeDtypeStruct((N, H), jnp.float32),
            jax.ShapeDtypeStruct((N, H), jnp.float32),
        ],
    )(x, wl_t, wr_t)


def _tc_combine_body(acc_ref, deg_ref, r_ref, bl_ref, wa_ref, wb_ref,
                     outa_ref, outb_ref):
    deg = deg_ref[0, :, 0] + deg_ref[1, :, 0]
    inv = 1.0 / jnp.maximum(deg, 1.0)
    agg = acc_ref[0] + acc_ref[1]
    h = jnp.maximum(agg * inv[:, None] + bl_ref[0] + r_ref[...], 0.0)
    outa_ref[...] = jnp.dot(h, wa_ref[...], preferred_element_type=jnp.float32)
    outb_ref[...] = jnp.dot(h, wb_ref[...], preferred_element_type=jnp.float32)


def _tc_combine(acc, deg_acc, r, bl, wa_t, wb_t):
    return pl.pallas_call(
        _tc_combine_body,
        grid=(N // BN,),
        in_specs=[
            pl.BlockSpec((NC, BN, H), lambda i: (0, i, 0)),
            pl.BlockSpec((NC, BN, 16), lambda i: (0, i, 0)),
            pl.BlockSpec((BN, H), lambda i: (i, 0)),
            pl.BlockSpec((1, H), lambda i: (0, 0)),
            pl.BlockSpec((H, H), lambda i: (0, 0)),
            pl.BlockSpec((H, H), lambda i: (0, 0)),
        ],
        out_specs=[
            pl.BlockSpec((BN, H), lambda i: (i, 0)),
            pl.BlockSpec((BN, H), lambda i: (i, 0)),
        ],
        out_shape=[
            jax.ShapeDtypeStruct((N, H), jnp.float32),
            jax.ShapeDtypeStruct((N, H), jnp.float32),
        ],
    )(acc, deg_acc, r, bl, wa_t, wb_t)


OUT_COLS = 2000
OUT_ROWS_PER_BLK = 8
BE = OUT_COLS * OUT_ROWS_PER_BLK


def _tc_edge_mlp_body(g_ref, ea_ref, w1e_ref, b1_ref, w2_ref, b2_ref,
                      w3_ref, b3_ref, out_ref):
    z = g_ref[...] + jnp.dot(ea_ref[...], w1e_ref[...],
                             preferred_element_type=jnp.float32) + b1_ref[0]
    z = jnp.maximum(z, 0.0)
    z = jnp.maximum(jnp.dot(z, w2_ref[...],
                            preferred_element_type=jnp.float32) + b2_ref[0], 0.0)
    lg = jnp.dot(z, w3_ref[...], preferred_element_type=jnp.float32) + b3_ref[0]
    out_ref[...] = lg.reshape(OUT_ROWS_PER_BLK, OUT_COLS)


def _tc_edge_mlp(g, edge_attr, w1e_t, b1, w2_t, b2, w3_t, b3):
    out = pl.pallas_call(
        _tc_edge_mlp_body,
        grid=(E // BE,),
        in_specs=[
            pl.BlockSpec((BE, H), lambda i: (i, 0)),
            pl.BlockSpec((BE, 16), lambda i: (i, 0)),
            pl.BlockSpec((16, H), lambda i: (0, 0)),
            pl.BlockSpec((1, H), lambda i: (0, 0)),
            pl.BlockSpec((H, 32), lambda i: (0, 0)),
            pl.BlockSpec((1, 32), lambda i: (0, 0)),
            pl.BlockSpec((32, 1), lambda i: (0, 0)),
            pl.BlockSpec((1, 1), lambda i: (0, 0)),
        ],
        out_specs=pl.BlockSpec((OUT_ROWS_PER_BLK, OUT_COLS), lambda i: (i, 0)),
        out_shape=jax.ShapeDtypeStruct((E // OUT_COLS, OUT_COLS), jnp.float32),
    )(g, edge_attr, w1e_t, b1, w2_t, b2, w3_t, b3)
    return out.reshape(E)


def kernel(x, edge_index, edge_attr, Wl1, bl1, Wr1, Wl2, bl2, Wr2,
           Wl3, bl3, Wr3, W1, b1, W2, b2, W3, b3):
    src = edge_index[0]
    dst = edge_index[1]

    zeros_n64 = jnp.zeros((NP, H), jnp.float32)
    zeros_n16 = jnp.zeros((NP, 16), jnp.float32)
    ones_c16 = jnp.ones((CHUNK, 16), jnp.float32)

    deg_acc = _sc_degree(dst, zeros_n16, ones_c16)

    pre1, r1 = _tc_proj(x, Wl1.T, Wr1.T)
    acc1 = _sc_segsum(pre1, src, dst, zeros_n64)
    pre2, r2 = _tc_combine(acc1, deg_acc, r1, bl1.reshape(1, H),
                           Wl2.T, Wr2.T)
    acc2 = _sc_segsum(pre2, src, dst, zeros_n64)
    pre3, r3 = _tc_combine(acc2, deg_acc, r2, bl2.reshape(1, H),
                           Wl3.T, Wr3.T)
    acc3 = _sc_segsum(pre3, src, dst, zeros_n64)
    w1s_t = W1[:, :H].T
    w1d_t = W1[:, H:2 * H].T
    hs, hd = _tc_combine(acc3, deg_acc, r3, bl3.reshape(1, H), w1s_t, w1d_t)

    g = _sc_edge_gather(hs, hd, src, dst)

    w1e_t = W1[:, 2 * H:].T
    return _tc_edge_mlp(g, edge_attr, w1e_t, b1.reshape(1, H),
                        W2.T, b2.reshape(1, 32), W3.T, b3.reshape(1, 1))


# speedup vs baseline: 8.3392x; 1.0536x over previous
import functools

import jax
import jax.numpy as jnp
from jax import lax
from jax.experimental import pallas as pl
from jax.experimental.pallas import tpu as pltpu
from jax.experimental.pallas import tpu_sc as plsc

N = 10000
E = 320000
H = 64

NC = 2
NS = 16
NW = NC * NS
EDGES_PER_W = E // NW
CHUNK = 400
NP = 10240
N_PER_TILE = NP // NS

_sc_mesh = plsc.VectorSubcoreMesh(core_axis_name="c", subcore_axis_name="s")


def _wid():
    return lax.axis_index("s") * NC + lax.axis_index("c")


N_TBL_SLAB = N // NS


@functools.partial(
    pl.kernel,
    out_type=jax.ShapeDtypeStruct((NC, NP, 16), jnp.float32),
    mesh=_sc_mesh,
    compiler_params=pltpu.CompilerParams(use_tc_tiling_on_sc=False),
    scratch_types=[
        pltpu.VMEM((CHUNK,), jnp.int32),
        pltpu.VMEM((CHUNK, 16), jnp.float32),
        pltpu.VMEM_SHARED((NP, 16), jnp.float32),
    ],
)
def _sc_degree(dst_hbm, zeros_hbm, ones_hbm, out_hbm, dst_v, ones_v, acc_sh):
    cid = lax.axis_index("c")
    sid = lax.axis_index("s")
    base = _wid() * EDGES_PER_W
    pltpu.sync_copy(zeros_hbm.at[pl.ds(sid * N_PER_TILE, N_PER_TILE)],
                    acc_sh.at[pl.ds(sid * N_PER_TILE, N_PER_TILE)])
    pltpu.sync_copy(ones_hbm, ones_v)
    plsc.subcore_barrier()

    def body(k, _):
        off = base + k * CHUNK
        pltpu.sync_copy(dst_hbm.at[pl.ds(off, CHUNK)], dst_v)
        pltpu.sync_copy(ones_v, acc_sh.at[dst_v], add=True)
        return 0

    lax.fori_loop(0, EDGES_PER_W // CHUNK, body, 0)
    plsc.subcore_barrier()
    pltpu.sync_copy(acc_sh.at[pl.ds(sid * N_PER_TILE, N_PER_TILE)],
                    out_hbm.at[cid, pl.ds(sid * N_PER_TILE, N_PER_TILE)])


K2 = (EDGES_PER_W // CHUNK - 1) // 2


@functools.partial(
    pl.kernel,
    out_type=jax.ShapeDtypeStruct((NC, NP, H), jnp.float32),
    mesh=_sc_mesh,
    compiler_params=pltpu.CompilerParams(use_tc_tiling_on_sc=False),
    scratch_types=[
        pltpu.VMEM((CHUNK,), jnp.int32),
        pltpu.VMEM((CHUNK,), jnp.int32),
        pltpu.VMEM((CHUNK,), jnp.int32),
        pltpu.VMEM((CHUNK,), jnp.int32),
        pltpu.VMEM((CHUNK, H), jnp.float32),
        pltpu.VMEM((CHUNK, H), jnp.float32),
        pltpu.VMEM_SHARED((NP, H), jnp.float32),
        pltpu.SemaphoreType.DMA,
        pltpu.SemaphoreType.DMA,
    ],
)
def _sc_segsum(pre_hbm, src_hbm, dst_hbm, zeros_hbm, out_hbm,
               src_a, dst_a, src_b, dst_b, rows_a, rows_b, acc_sh,
               sem_a, sem_b):
    cid = lax.axis_index("c")
    sid = lax.axis_index("s")
    base = _wid() * EDGES_PER_W
    pltpu.sync_copy(zeros_hbm.at[pl.ds(sid * N_PER_TILE, N_PER_TILE)],
                    acc_sh.at[pl.ds(sid * N_PER_TILE, N_PER_TILE)])
    plsc.subcore_barrier()

    def load(off, s_v, d_v):
        pltpu.sync_copy(src_hbm.at[pl.ds(off, CHUNK)], s_v)
        pltpu.sync_copy(dst_hbm.at[pl.ds(off, CHUNK)], d_v)

    load(base, src_a, dst_a)
    pltpu.async_copy(pre_hbm.at[src_a], rows_a, sem_a)

    def body(k2, _):
        load(base + (2 * k2 + 1) * CHUNK, src_b, dst_b)
        pltpu.async_copy(pre_hbm.at[src_b], rows_b, sem_b)
        pltpu.make_async_copy(pre_hbm.at[src_a], rows_a, sem_a).wait()
        pltpu.sync_copy(rows_a, acc_sh.at[dst_a], add=True)

        load(base + (2 * k2 + 2) * CHUNK, src_a, dst_a)
        pltpu.async_copy(pre_hbm.at[src_a], rows_a, sem_a)

        pltpu.make_async_copy(pre_hbm.at[src_b], rows_b, sem_b).wait()
        pltpu.sync_copy(rows_b, acc_sh.at[dst_b], add=True)
        return 0

    lax.fori_loop(0, K2, body, 0)
    pltpu.make_async_copy(pre_hbm.at[src_a], rows_a, sem_a).wait()
    pltpu.sync_copy(rows_a, acc_sh.at[dst_a], add=True)
    plsc.subcore_barrier()
    pltpu.sync_copy(acc_sh.at[pl.ds(sid * N_PER_TILE, N_PER_TILE)],
                    out_hbm.at[cid, pl.ds(sid * N_PER_TILE, N_PER_TILE)])


@functools.partial(
    pl.kernel,
    out_type=jax.ShapeDtypeStruct((E, H), jnp.float32),
    mesh=_sc_mesh,
    compiler_params=pltpu.CompilerParams(use_tc_tiling_on_sc=False),
    scratch_types=[
        pltpu.VMEM((CHUNK,), jnp.int32),
        pltpu.VMEM((CHUNK,), jnp.int32),
        pltpu.VMEM((CHUNK,), jnp.int32),
        pltpu.VMEM((CHUNK,), jnp.int32),
        pltpu.VMEM((CHUNK, H), jnp.float32),
        pltpu.VMEM((CHUNK, H), jnp.float32),
        pltpu.VMEM_SHARED((N, H), jnp.float32),
        pltpu.SemaphoreType.DMA,
        pltpu.SemaphoreType.DMA,
        pltpu.SemaphoreType.DMA,
        pltpu.SemaphoreType.DMA,
    ],
)
def _sc_edge_gather(hs_hbm, hd_hbm, src_hbm, dst_hbm, out_hbm,
                    src_a, dst_a, src_b, dst_b, rows_a, rows_b, hs_sh,
                    sem_a1, sem_a2, sem_b1, sem_b2):
    sid = lax.axis_index("s")
    base = _wid() * EDGES_PER_W
    pltpu.sync_copy(hs_hbm.at[pl.ds(sid * N_TBL_SLAB, N_TBL_SLAB)],
                    hs_sh.at[pl.ds(sid * N_TBL_SLAB, N_TBL_SLAB)])
    plsc.subcore_barrier()

    def load(off, s_v, d_v):
        pltpu.sync_copy(src_hbm.at[pl.ds(off, CHUNK)], s_v)
        pltpu.sync_copy(dst_hbm.at[pl.ds(off, CHUNK)], d_v)

    load(base, src_a, dst_a)
    pltpu.async_copy(hs_sh.at[src_a], rows_a, sem_a1)

    def body(k2, _):
        off_a = base + 2 * k2 * CHUNK
        off_b = off_a + CHUNK
        load(off_b, src_b, dst_b)
        pltpu.async_copy(hs_sh.at[src_b], rows_b, sem_b1)
        pltpu.make_async_copy(hs_sh.at[src_a], rows_a, sem_a1).wait()
        pltpu.async_copy(hd_hbm.at[dst_a], rows_a, sem_a2, add=True).wait()
        pltpu.sync_copy(rows_a, out_hbm.at[pl.ds(off_a, CHUNK)])

        load(off_b + CHUNK, src_a, dst_a)
        pltpu.async_copy(hs_sh.at[src_a], rows_a, sem_a1)

        pltpu.make_async_copy(hs_sh.at[src_b], rows_b, sem_b1).wait()
        pltpu.async_copy(hd_hbm.at[dst_b], rows_b, sem_b2, add=True).wait()
        pltpu.sync_copy(rows_b, out_hbm.at[pl.ds(off_b, CHUNK)])
        return 0

    lax.fori_loop(0, K2, body, 0)
    off_last = base + EDGES_PER_W - CHUNK
    pltpu.make_async_copy(hs_sh.at[src_a], rows_a, sem_a1).wait()
    pltpu.async_copy(hd_hbm.at[dst_a], rows_a, sem_a2, add=True).wait()
    pltpu.sync_copy(rows_a, out_hbm.at[pl.ds(off_last, CHUNK)])


BN = 1000


def _tc_proj_body(x_ref, wl_ref, wr_ref, pre_ref, r_ref):
    x = x_ref[...]
    pre_ref[...] = jnp.dot(x, wl_ref[...], preferred_element_type=jnp.float32)
    r_ref[...] = jnp.dot(x, wr_ref[...], preferred_element_type=jnp.float32)


def _tc_proj(x, wl_t, wr_t):
    d = x.shape[1]
    return pl.pallas_call(
        _tc_proj_body,
        grid=(N // BN,),
        in_specs=[
            pl.BlockSpec((BN, d), lambda i: (i, 0)),
            pl.BlockSpec((d, H), lambda i: (0, 0)),
            pl.BlockSpec((d, H), lambda i: (0, 0)),
        ],
        out_specs=[
            pl.BlockSpec((BN, H), lambda i: (i, 0)),
            pl.BlockSpec((BN, H), lambda i: (i, 0)),
        ],
        out_shape=[
            jax.ShapeDtypeStruct((N, H), jnp.float32),
            jax.ShapeDtypeStruct((N, H), jnp.float32),
        ],
    )(x, wl_t, wr_t)


def _tc_combine_body(acc_ref, deg_ref, r_ref, bl_ref, wa_ref, wb_ref,
                     outa_ref, outb_ref):
    deg = deg_ref[0, :, 0] + deg_ref[1, :, 0]
    inv = 1.0 / jnp.maximum(deg, 1.0)
    agg = acc_ref[0] + acc_ref[1]
    h = jnp.maximum(agg * inv[:, None] + bl_ref[0] + r_ref[...], 0.0)
    outa_ref[...] = jnp.dot(h, wa_ref[...], preferred_element_type=jnp.float32)
    outb_ref[...] = jnp.dot(h, wb_ref[...], preferred_element_type=jnp.float32)


def _tc_combine(acc, deg_acc, r, bl, wa_t, wb_t):
    return pl.pallas_call(
        _tc_combine_body,
        grid=(N // BN,),
        in_specs=[
            pl.BlockSpec((NC, BN, H), lambda i: (0, i, 0)),
            pl.BlockSpec((NC, BN, 16), lambda i: (0, i, 0)),
            pl.BlockSpec((BN, H), lambda i: (i, 0)),
            pl.BlockSpec((1, H), lambda i: (0, 0)),
            pl.BlockSpec((H, H), lambda i: (0, 0)),
            pl.BlockSpec((H, H), lambda i: (0, 0)),
        ],
        out_specs=[
            pl.BlockSpec((BN, H), lambda i: (i, 0)),
            pl.BlockSpec((BN, H), lambda i: (i, 0)),
        ],
        out_shape=[
            jax.ShapeDtypeStruct((N, H), jnp.float32),
            jax.ShapeDtypeStruct((N, H), jnp.float32),
        ],
    )(acc, deg_acc, r, bl, wa_t, wb_t)


OUT_COLS = 2000
OUT_ROWS_PER_BLK = 8
BE = OUT_COLS * OUT_ROWS_PER_BLK


def _tc_edge_mlp_body(g_ref, ea_ref, w1e_ref, b1_ref, w2_ref, b2_ref,
                      w3_ref, b3_ref, out_ref):
    z = g_ref[...] + jnp.dot(ea_ref[...], w1e_ref[...],
                             preferred_element_type=jnp.float32) + b1_ref[0]
    z = jnp.maximum(z, 0.0)
    z = jnp.maximum(jnp.dot(z, w2_ref[...],
                            preferred_element_type=jnp.float32) + b2_ref[0], 0.0)
    lg = jnp.dot(z, w3_ref[...], preferred_element_type=jnp.float32) + b3_ref[0]
    out_ref[...] = lg.reshape(OUT_ROWS_PER_BLK, OUT_COLS)


def _tc_edge_mlp(g, edge_attr, w1e_t, b1, w2_t, b2, w3_t, b3):
    out = pl.pallas_call(
        _tc_edge_mlp_body,
        grid=(E // BE,),
        in_specs=[
            pl.BlockSpec((BE, H), lambda i: (i, 0)),
            pl.BlockSpec((BE, 16), lambda i: (i, 0)),
            pl.BlockSpec((16, H), lambda i: (0, 0)),
            pl.BlockSpec((1, H), lambda i: (0, 0)),
            pl.BlockSpec((H, 32), lambda i: (0, 0)),
            pl.BlockSpec((1, 32), lambda i: (0, 0)),
            pl.BlockSpec((32, 1), lambda i: (0, 0)),
            pl.BlockSpec((1, 1), lambda i: (0, 0)),
        ],
        out_specs=pl.BlockSpec((OUT_ROWS_PER_BLK, OUT_COLS), lambda i: (i, 0)),
        out_shape=jax.ShapeDtypeStruct((E // OUT_COLS, OUT_COLS), jnp.float32),
    )(g, edge_attr, w1e_t, b1, w2_t, b2, w3_t, b3)
    return out.reshape(E)


def kernel(x, edge_index, edge_attr, Wl1, bl1, Wr1, Wl2, bl2, Wr2,
           Wl3, bl3, Wr3, W1, b1, W2, b2, W3, b3):
    src = edge_index[0]
    dst = edge_index[1]

    zeros_n64 = jnp.zeros((NP, H), jnp.float32)
    zeros_n16 = jnp.zeros((NP, 16), jnp.float32)
    ones_c16 = jnp.ones((CHUNK, 16), jnp.float32)

    deg_acc = _sc_degree(dst, zeros_n16, ones_c16)

    pre1, r1 = _tc_proj(x, Wl1.T, Wr1.T)
    acc1 = _sc_segsum(pre1, src, dst, zeros_n64)
    pre2, r2 = _tc_combine(acc1, deg_acc, r1, bl1.reshape(1, H),
                           Wl2.T, Wr2.T)
    acc2 = _sc_segsum(pre2, src, dst, zeros_n64)
    pre3, r3 = _tc_combine(acc2, deg_acc, r2, bl2.reshape(1, H),
                           Wl3.T, Wr3.T)
    acc3 = _sc_segsum(pre3, src, dst, zeros_n64)
    w1s_t = W1[:, :H].T
    w1d_t = W1[:, H:2 * H].T
    hs, hd = _tc_combine(acc3, deg_acc, r3, bl3.reshape(1, H), w1s_t, w1d_t)

    g = _sc_edge_gather(hs, hd, src, dst)

    w1e_t = W1[:, 2 * H:].T
    return _tc_edge_mlp(g, edge_attr, w1e_t, b1.reshape(1, H),
                        W2.T, b2.reshape(1, 32), W3.T, b3.reshape(1, 1))
